# Initial kernel scaffold; baseline (speedup 1.0000x reference)
import jax
import jax.numpy as jnp
from jax.experimental import pallas as pl


def kernel(x, edge_index, W1, W2, W3):
    raise NotImplementedError("write your pallas kernel here")


# speedup vs baseline: 4.8237x; 4.8237x over previous
import functools

import jax
import jax.numpy as jnp
from jax import lax
from jax.experimental import pallas as pl
from jax.experimental.pallas import tpu as pltpu
from jax.experimental.pallas import tpu_sc as plsc

N = 10000
NPAD = 10240
E = 160000
D = 256
H = 128
NC = 2
NS = 16
BN = 1000
NB = N // BN

_F32 = jnp.float32


def _sc_mesh():
    return plsc.VectorSubcoreMesh(core_axis_name="c", subcore_axis_name="s")


_DEG_CH = 40
_DEG_EPW = E // (NC * NS)
_DEG_NCH = _DEG_EPW // _DEG_CH
_ROWS_W16 = NPAD // NS


@functools.partial(
    pl.kernel,
    out_type=jax.ShapeDtypeStruct((NC * NPAD, 16), _F32),
    mesh=_sc_mesh(),
    scratch_types=[
        pltpu.VMEM((_DEG_CH,), jnp.int32),
        pltpu.VMEM((_DEG_CH, 16), _F32),
        pltpu.VMEM_SHARED((NPAD, 16), _F32),
    ],
)
def _sc_degree(dst_hbm, zeros_hbm, out_hbm, dst_v, ones_v, acc_sh):
    c = lax.axis_index("c")
    s = lax.axis_index("s")
    for i in range(_DEG_CH):
        ones_v[i] = jnp.full((16,), 1.0, _F32)
    pltpu.sync_copy(zeros_hbm.at[pl.ds(s * _ROWS_W16, _ROWS_W16)],
                    acc_sh.at[pl.ds(s * _ROWS_W16, _ROWS_W16)])
    plsc.subcore_barrier()
    base = (c * NS + s) * _DEG_EPW

    @pl.loop(0, _DEG_NCH)
    def _(j):
        pltpu.sync_copy(dst_hbm.at[pl.ds(base + j * _DEG_CH, _DEG_CH)], dst_v)
        pltpu.sync_copy(ones_v, acc_sh.at[dst_v], add=True)

    plsc.subcore_barrier()
    pltpu.sync_copy(acc_sh.at[pl.ds(s * _ROWS_W16, _ROWS_W16)],
                    out_hbm.at[pl.ds(c * NPAD + s * _ROWS_W16, _ROWS_W16)])


_SP_CH = 80
_SP_EPW = E // NS
_SP_NCH = _SP_EPW // _SP_CH
_ROWS_WH = NPAD // NS


@functools.partial(
    pl.kernel,
    out_type=jax.ShapeDtypeStruct((NC * NPAD, H), _F32),
    mesh=_sc_mesh(),
    scratch_types=[
        pltpu.VMEM((_SP_CH,), jnp.int32),
        pltpu.VMEM((_SP_CH,), jnp.int32),
        pltpu.VMEM((_SP_CH,), jnp.int32),
        pltpu.VMEM((_SP_CH, H), _F32),
        pltpu.VMEM_SHARED((NPAD, H), _F32),
        pltpu.SemaphoreType.DMA,
    ],
)
def _sc_spmm(hs_hbm, src_hbm, dst_hbm, zeros_hbm, out_hbm,
             src_v, gidx_v, dst_v, rows_v, acc_sh, sem):
    c = lax.axis_index("c")
    s = lax.axis_index("s")
    pltpu.sync_copy(zeros_hbm.at[pl.ds(s * _ROWS_WH, _ROWS_WH)],
                    acc_sh.at[pl.ds(s * _ROWS_WH, _ROWS_WH)])
    plsc.subcore_barrier()
    base = s * _SP_EPW
    off = c * NPAD

    @pl.loop(0, _SP_NCH)
    def _(j):
        e0 = base + j * _SP_CH
        pltpu.sync_copy(src_hbm.at[pl.ds(e0, _SP_CH)], src_v)
        pltpu.sync_copy(dst_hbm.at[pl.ds(e0, _SP_CH)], dst_v)

        @pl.loop(0, _SP_CH, step=16)
        def _(i):
            gidx_v[pl.ds(i, 16)] = src_v[pl.ds(i, 16)] + off

        pltpu.async_copy(hs_hbm.at[gidx_v], rows_v, sem).wait()
        pltpu.sync_copy(rows_v, acc_sh.at[dst_v], add=True)

    plsc.subcore_barrier()
    pltpu.sync_copy(acc_sh.at[pl.ds(s * _ROWS_WH, _ROWS_WH)],
                    out_hbm.at[pl.ds(off + s * _ROWS_WH, _ROWS_WH)])


def _dot(a, b, dims):
    return lax.dot_general(a, b, (dims, ((), ())),
                           precision=lax.Precision.HIGHEST,
                           preferred_element_type=_F32)


def _tc_prep_body(deg0_ref, deg1_ref, x_ref, d_ref, xs_ref):
    deg = deg0_ref[0, :, 0:1] + deg1_ref[0, :, 0:1] + 1.0
    d = lax.rsqrt(deg)
    d_ref[...] = d
    xs_ref[0] = x_ref[...] * d


def _tc_prep(deg2, x):
    return pl.pallas_call(
        _tc_prep_body,
        grid=(NB, NC),
        in_specs=[
            pl.BlockSpec((1, BN, 16), lambda i, c: (0, i, 0)),
            pl.BlockSpec((1, BN, 16), lambda i, c: (1, i, 0)),
            pl.BlockSpec((BN, H), lambda i, c: (i, c)),
        ],
        out_specs=[
            pl.BlockSpec((BN, 1), lambda i, c: (i, 0)),
            pl.BlockSpec((1, BN, H), lambda i, c: (c, i, 0)),
        ],
        out_shape=[
            jax.ShapeDtypeStruct((N, 1), _F32),
            jax.ShapeDtypeStruct((NC, NPAD, H), _F32),
        ],
    )(deg2, deg2, x)


def _tc_layer_body(t0_ref, t1_ref, h0_ref, h1_ref, d_ref, w_ref, ys_ref):
    d = d_ref[...]
    u = jnp.concatenate(
        [t0_ref[0] + h0_ref[0], t1_ref[0] + h1_ref[0]], axis=1) * d
    y = jnp.maximum(_dot(u, w_ref[...], ((1,), (0,))), 0.0)
    ys_ref[0] = y * d


def _tc_layer(t, hs, d, W):
    return pl.pallas_call(
        _tc_layer_body,
        grid=(NB, NC),
        in_specs=[
            pl.BlockSpec((1, BN, H), lambda i, c: (0, i, 0)),
            pl.BlockSpec((1, BN, H), lambda i, c: (1, i, 0)),
            pl.BlockSpec((1, BN, H), lambda i, c: (0, i, 0)),
            pl.BlockSpec((1, BN, H), lambda i, c: (1, i, 0)),
            pl.BlockSpec((BN, 1), lambda i, c: (i, 0)),
            pl.BlockSpec((D, H), lambda i, c: (0, c)),
        ],
        out_specs=pl.BlockSpec((1, BN, H), lambda i, c: (c, i, 0)),
        out_shape=jax.ShapeDtypeStruct((NC, NPAD, H), _F32),
    )(t, t, hs, hs, d, W)


def _tc_final_body(t0_ref, t1_ref, h0_ref, h1_ref, d_ref, w_ref, z_ref):
    u = jnp.concatenate(
        [t0_ref[0] + h0_ref[0], t1_ref[0] + h1_ref[0]], axis=1) * d_ref[...]
    z_ref[...] = _dot(u, w_ref[...], ((1,), (0,)))


def _tc_final(t, hs, d, W):
    return pl.pallas_call(
        _tc_final_body,
        grid=(NB, NC),
        in_specs=[
            pl.BlockSpec((1, BN, H), lambda i, c: (0, i, 0)),
            pl.BlockSpec((1, BN, H), lambda i, c: (1, i, 0)),
            pl.BlockSpec((1, BN, H), lambda i, c: (0, i, 0)),
            pl.BlockSpec((1, BN, H), lambda i, c: (1, i, 0)),
            pl.BlockSpec((BN, 1), lambda i, c: (i, 0)),
            pl.BlockSpec((D, H), lambda i, c: (0, c)),
        ],
        out_specs=pl.BlockSpec((BN, H), lambda i, c: (i, c)),
        out_shape=jax.ShapeDtypeStruct((N, D), _F32),
    )(t, t, hs, hs, d, W)


def _tc_cos_body(z_ref, o_ref):
    z = z_ref[...]
    zn = z * lax.rsqrt(jnp.sum(z * z, axis=1, keepdims=True))
    g = _dot(zn, zn, ((1,), (1,)))
    o_ref[...] = (jnp.sum(g) * (1.0 / (512.0 * 512.0))).reshape(1, 1)


def _tc_cos(z512):
    return pl.pallas_call(
        _tc_cos_body,
        out_shape=jax.ShapeDtypeStruct((1, 1), _F32),
    )(z512)


def kernel(x, edge_index, W1, W2, W3):
    src = edge_index[0]
    dst = edge_index[1]
    zeros16 = jnp.zeros((NPAD, 16), _F32)
    zerosH = jnp.zeros((NPAD, H), _F32)

    deg2 = _sc_degree(dst, zeros16).reshape(NC, NPAD, 16)
    d, xs = _tc_prep(deg2, x)

    t1 = _sc_spmm(xs.reshape(NC * NPAD, H), src, dst, zerosH).reshape(NC, NPAD, H)
    h1s = _tc_layer(t1, xs, d, W1)

    t2 = _sc_spmm(h1s.reshape(NC * NPAD, H), src, dst, zerosH).reshape(NC, NPAD, H)
    h2s = _tc_layer(t2, h1s, d, W2)

    t3 = _sc_spmm(h2s.reshape(NC * NPAD, H), src, dst, zerosH).reshape(NC, NPAD, H)
    z = _tc_final(t3, h2s, d, W3)

    corr = _tc_cos(z[:512])
    return z, corr[0, 0]


# speedup vs baseline: 5.0764x; 1.0524x over previous
import functools

import jax
import jax.numpy as jnp
from jax import lax
from jax.experimental import pallas as pl
from jax.experimental.pallas import tpu as pltpu
from jax.experimental.pallas import tpu_sc as plsc

N = 10000
NPAD = 10240
E = 160000
EPAD = 163840
D = 256
H = 128
NC = 2
NS = 16
BN = 1000
NB = N // BN

_F32 = jnp.float32
_CH = 128
_ROWS_W = NPAD // NS


def _sc_mesh():
    return plsc.VectorSubcoreMesh(core_axis_name="c", subcore_axis_name="s")


_DEG_NCH = EPAD // (NC * NS) // _CH
_DEG_WIN = 8


@functools.partial(
    pl.kernel,
    out_type=jax.ShapeDtypeStruct((NC * NPAD, 16), _F32),
    mesh=_sc_mesh(),
    scratch_types=[
        pltpu.VMEM((_DEG_NCH, _CH), jnp.int32),
        pltpu.VMEM((_CH, 16), _F32),
        pltpu.VMEM_SHARED((NPAD, 16), _F32),
        pltpu.SemaphoreType.DMA,
    ],
)
def _sc_degree(dst_hbm, zeros_hbm, out_hbm, dst_all, ones_v, acc_sh, sem):
    c = lax.axis_index("c")
    s = lax.axis_index("s")
    w = c * NS + s
    pltpu.sync_copy(dst_hbm.at[w], dst_all)
    for i in range(_CH):
        ones_v[i] = jnp.full((16,), 1.0, _F32)
    pltpu.sync_copy(zeros_hbm.at[pl.ds(s * _ROWS_W, _ROWS_W)],
                    acc_sh.at[pl.ds(s * _ROWS_W, _ROWS_W)])
    plsc.subcore_barrier()

    @pl.loop(0, _DEG_NCH)
    def _(j):
        pltpu.async_copy(ones_v, acc_sh.at[dst_all.at[j]], sem, add=True)

        @pl.when(j >= _DEG_WIN)
        def _():
            pltpu.make_async_copy(ones_v, acc_sh.at[dst_all.at[0]], sem).wait()

    @pl.loop(0, _DEG_WIN)
    def _(j):
        pltpu.make_async_copy(ones_v, acc_sh.at[dst_all.at[0]], sem).wait()

    plsc.subcore_barrier()
    pltpu.sync_copy(acc_sh.at[pl.ds(s * _ROWS_W, _ROWS_W)],
                    out_hbm.at[pl.ds(c * NPAD + s * _ROWS_W, _ROWS_W)])


_SP_NCH = EPAD // NS // _CH
_SP_NBLK = _SP_NCH // 8


@functools.partial(
    pl.kernel,
    out_type=[jax.ShapeDtypeStruct((NPAD, H), _F32),
              jax.ShapeDtypeStruct((NPAD, H), _F32)],
    mesh=_sc_mesh(),
    scratch_types=[
        pltpu.VMEM((_SP_NCH, _CH), jnp.int32),
        pltpu.VMEM((16, _CH), jnp.int32),
        pltpu.VMEM((2 * _CH, H), _F32),
        pltpu.VMEM_SHARED((NPAD, H), _F32),
        pltpu.SemaphoreType.DMA,
        pltpu.SemaphoreType.DMA,
    ],
)
def _sc_spmm(hs0_hbm, hs1_hbm, src_hbm, dst_hbm, zeros_hbm,
             out0_hbm, out1_hbm, dst_all, ring, rows, acc_sh, sem, semi):
    c = lax.axis_index("c")
    s = lax.axis_index("s")
    pltpu.sync_copy(dst_hbm.at[s], dst_all)
    pltpu.sync_copy(zeros_hbm.at[pl.ds(s * _ROWS_W, _ROWS_W)],
                    acc_sh.at[pl.ds(s * _ROWS_W, _ROWS_W)])

    def _gather(j, ridx):
        @pl.when(c == 0)
        def _():
            pltpu.async_copy(hs0_hbm.at[ring.at[ridx]],
                             rows.at[pl.ds(lax.rem(j, 2) * _CH, _CH)], sem)

        @pl.when(c == 1)
        def _():
            pltpu.async_copy(hs1_hbm.at[ring.at[ridx]],
                             rows.at[pl.ds(lax.rem(j, 2) * _CH, _CH)], sem)

    pltpu.sync_copy(src_hbm.at[s].at[0], ring.at[pl.ds(0, 8)])
    pltpu.async_copy(src_hbm.at[s].at[1], ring.at[pl.ds(8, 8)], semi)
    plsc.subcore_barrier()
    _gather(0, 0)

    @pl.loop(0, _SP_NCH)
    def _(j):
        pltpu.make_async_copy(hs0_hbm.at[ring.at[0]],
                              rows.at[pl.ds(0, _CH)], sem).wait()

        j1 = j + 1

        @pl.when(j1 < _SP_NCH)
        def _():
            b1 = j1 // 8

            @pl.when(lax.rem(j1, 8) == 0)
            def _():
                pltpu.make_async_copy(src_hbm.at[s].at[0],
                                      ring.at[pl.ds(0, 8)], semi).wait()

                @pl.when(b1 + 1 < _SP_NBLK)
                def _():
                    pltpu.async_copy(
                        src_hbm.at[s].at[b1 + 1],
                        ring.at[pl.ds(lax.rem(b1 + 1, 2) * 8, 8)], semi)

            _gather(j1, lax.rem(b1, 2) * 8 + lax.rem(j1, 8))

        pltpu.sync_copy(rows.at[pl.ds(lax.rem(j, 2) * _CH, _CH)],
                        acc_sh.at[dst_all.at[j]], add=True)

    plsc.subcore_barrier()

    @pl.when(c == 0)
    def _():
        pltpu.sync_copy(acc_sh.at[pl.ds(s * _ROWS_W, _ROWS_W)],
                        out0_hbm.at[pl.ds(s * _ROWS_W, _ROWS_W)])

    @pl.when(c == 1)
    def _():
        pltpu.sync_copy(acc_sh.at[pl.ds(s * _ROWS_W, _ROWS_W)],
                        out1_hbm.at[pl.ds(s * _ROWS_W, _ROWS_W)])


def _dot(a, b, dims):
    return lax.dot_general(a, b, (dims, ((), ())),
                           precision=lax.Precision.HIGHEST,
                           preferred_element_type=_F32)


def _half_spec():
    return pl.BlockSpec((BN, H), lambda i: (i, 0))


def _tc_prep_body(deg0_ref, deg1_ref, x_ref, d_ref, xs0_ref, xs1_ref):
    deg = deg0_ref[0, :, 0:1] + deg1_ref[0, :, 0:1] + 1.0
    d = lax.rsqrt(deg)
    d_ref[...] = d
    xs = x_ref[...] * d
    xs0_ref[...] = xs[:, :H]
    xs1_ref[...] = xs[:, H:]


def _tc_prep(deg2, x):
    return pl.pallas_call(
        _tc_prep_body,
        grid=(NB,),
        in_specs=[
            pl.BlockSpec((1, BN, 16), lambda i: (0, i, 0)),
            pl.BlockSpec((1, BN, 16), lambda i: (1, i, 0)),
            pl.BlockSpec((BN, D), lambda i: (i, 0)),
        ],
        out_specs=[pl.BlockSpec((BN, 1), lambda i: (i, 0)),
                   _half_spec(), _half_spec()],
        out_shape=[jax.ShapeDtypeStruct((N, 1), _F32),
                   jax.ShapeDtypeStruct((NPAD, H), _F32),
                   jax.ShapeDtypeStruct((NPAD, H), _F32)],
    )(deg2, deg2, x)


def _tc_layer_body(t0_ref, t1_ref, h0_ref, h1_ref, d_ref, w_ref,
                   ys0_ref, ys1_ref):
    d = d_ref[...]
    u = jnp.concatenate(
        [t0_ref[...] + h0_ref[...], t1_ref[...] + h1_ref[...]], axis=1) * d
    y = jnp.maximum(_dot(u, w_ref[...], ((1,), (0,))), 0.0) * d
    ys0_ref[...] = y[:, :H]
    ys1_ref[...] = y[:, H:]


def _tc_layer(t0, t1, h0, h1, d, W):
    return pl.pallas_call(
        _tc_layer_body,
        grid=(NB,),
        in_specs=[_half_spec(), _half_spec(), _half_spec(), _half_spec(),
                  pl.BlockSpec((BN, 1), lambda i: (i, 0)),
                  pl.BlockSpec((D, D), lambda i: (0, 0))],
        out_specs=[_half_spec(), _half_spec()],
        out_shape=[jax.ShapeDtypeStruct((NPAD, H), _F32),
                   jax.ShapeDtypeStruct((NPAD, H), _F32)],
    )(t0, t1, h0, h1, d, W)


def _tc_final_body(t0_ref, t1_ref, h0_ref, h1_ref, d_ref, w_ref, z_ref):
    u = jnp.concatenate(
        [t0_ref[...] + h0_ref[...], t1_ref[...] + h1_ref[...]],
        axis=1) * d_ref[...]
    z_ref[...] = _dot(u, w_ref[...], ((1,), (0,)))


def _tc_final(t0, t1, h0, h1, d, W):
    return pl.pallas_call(
        _tc_final_body,
        grid=(NB,),
        in_specs=[_half_spec(), _half_spec(), _half_spec(), _half_spec(),
                  pl.BlockSpec((BN, 1), lambda i: (i, 0)),
                  pl.BlockSpec((D, D), lambda i: (0, 0))],
        out_specs=pl.BlockSpec((BN, D), lambda i: (i, 0)),
        out_shape=jax.ShapeDtypeStruct((N, D), _F32),
    )(t0, t1, h0, h1, d, W)


def _tc_cos_body(z_ref, o_ref):
    z = z_ref[...]
    zn = z * lax.rsqrt(jnp.sum(z * z, axis=1, keepdims=True))
    g = _dot(zn, zn, ((1,), (1,)))
    o_ref[...] = (jnp.sum(g) * (1.0 / (512.0 * 512.0))).reshape(1, 1)


def _tc_cos(z512):
    return pl.pallas_call(
        _tc_cos_body,
        out_shape=jax.ShapeDtypeStruct((1, 1), _F32),
    )(z512)


def kernel(x, edge_index, W1, W2, W3):
    npad_e = EPAD - E
    src = jnp.concatenate([edge_index[0], jnp.zeros((npad_e,), jnp.int32)])
    dst = jnp.concatenate([edge_index[1],
                           jnp.full((npad_e,), N, jnp.int32)])
    srcR = src.reshape(NS, _SP_NBLK, 8, _CH)
    dstR = dst.reshape(NS, _SP_NCH, _CH)
    dst_degR = dst.reshape(NC * NS, _DEG_NCH, _CH)
    zeros16 = jnp.zeros((NPAD, 16), _F32)
    zerosH = jnp.zeros((NPAD, H), _F32)

    deg2 = _sc_degree(dst_degR, zeros16).reshape(NC, NPAD, 16)
    d, xs0, xs1 = _tc_prep(deg2, x)

    t0, t1 = _sc_spmm(xs0, xs1, srcR, dstR, zerosH)
    h0, h1 = _tc_layer(t0, t1, xs0, xs1, d, W1)

    t0, t1 = _sc_spmm(h0, h1, srcR, dstR, zerosH)
    g0, g1 = _tc_layer(t0, t1, h0, h1, d, W2)

    t0, t1 = _sc_spmm(g0, g1, srcR, dstR, zerosH)
    z = _tc_final(t0, t1, g0, g1, d, W3)

    corr = _tc_cos(z[:512])
    return z, corr[0, 0]
